# baseline (speedup 1.0000x reference)
import functools

import jax
import jax.numpy as jnp
from jax import lax
from jax.experimental import pallas as pl
from jax.experimental.pallas import tpu as pltpu
from jax.experimental.pallas import tpu_sc as plsc

_D = 64
_BT = 128
_NBUF = 4
_NC = 2
_NS = 16
_NW = _NC * _NS
_L = 16


def _k1_body(tin_ref, tout_ref):
    blk = tin_ref[...]
    t = blk.T
    t4 = t.reshape(256, 2, 64)
    tout_ref[...] = jnp.concatenate([t4[:, 0, :], t4[:, 1, :]], axis=1)


def _table_rowmajor(tT):
    return pl.pallas_call(
        _k1_body,
        grid=(1954,),
        in_specs=[pl.BlockSpec((64, 512), lambda i: (0, i))],
        out_specs=pl.BlockSpec((256, 128), lambda i: (i, 0)),
        out_shape=jax.ShapeDtypeStruct((500000, 128), jnp.float32),
    )(tT)


def _k2_body(xT_hbm, tbl_hbm, out_hbm, idx_all, *rest, hist):
    gbufs = rest[:_NBUF]
    tbufs = rest[_NBUF:2 * _NBUF]
    gsems = rest[2 * _NBUF:3 * _NBUF]
    ssems = rest[3 * _NBUF:4 * _NBUF]

    wid = lax.axis_index("s") * _NC + lax.axis_index("c")

    pltpu.sync_copy(xT_hbm.at[:, pl.ds(wid * _BT, _BT)], idx_all)

    def fire_gather(b, h):
        pltpu.async_copy(tbl_hbm.at[idx_all.at[h]], gbufs[b], gsems[b])

    def wait_gather(b, h):
        pltpu.make_async_copy(tbl_hbm.at[idx_all.at[h]], gbufs[b],
                              gsems[b]).wait()

    def fire_store(b, h):
        pltpu.async_copy(tbufs[b], out_hbm.at[h, :, wid], ssems[b])

    def wait_store(b, h):
        pltpu.make_async_copy(tbufs[b], out_hbm.at[h, :, wid],
                              ssems[b]).wait()

    def transpose_fix(b, h):
        gbuf, tbuf = gbufs[b], tbufs[b]
        masks = []
        rows = []
        base = lax.iota(jnp.int32, _L)
        for k in range(_BT // _L):
            ivec = idx_all[h, pl.ds(k * _L, _L)]
            masks.append(ivec == 0)
            rows.append(base + k * _L)
        zeros = jnp.zeros((_L,), jnp.float32)

        def dblock(dt, carry):
            colbase = jnp.zeros((_L,), jnp.int32) + dt * 8
            for ds in range(8):
                col = colbase + ds
                for k in range(_BT // _L):
                    v = plsc.load_gather(gbuf, [rows[k], col])
                    v = jnp.where(masks[k], zeros, v)
                    tbuf[dt, ds, pl.ds(k * _L, _L)] = v
            return carry

        lax.fori_loop(0, 8, dblock, 0)

    for b in range(_NBUF):
        fire_gather(b, b)

    def outer(i, carry):
        for b in range(_NBUF):
            h = i * _NBUF + b
            wait_gather(b, h)
            transpose_fix(b, h)
            fire_store(b, h)

            @pl.when(h + _NBUF < hist)
            def _():
                wait_store(b, h)
                fire_gather(b, h + _NBUF)
        return carry

    lax.fori_loop(0, hist // _NBUF, outer, 0)

    for b in range(_NBUF):
        wait_store(b, hist - _NBUF + b)


def kernel(X, table):
    batch, hist = X.shape
    vocab = table.shape[0]
    n_bt = batch // _BT

    tT = table.T
    tbl = _table_rowmajor(tT).reshape(vocab, _D)
    xT = X.T

    mesh = plsc.VectorSubcoreMesh(core_axis_name="c", subcore_axis_name="s",
                                  num_cores=_NC, num_subcores=_NS)
    scratch = (
        [pltpu.VMEM((hist, _BT), jnp.int32)]
        + [pltpu.VMEM((_BT, _D), jnp.float32)] * _NBUF
        + [pltpu.VMEM((8, 8, _BT), jnp.float32)] * _NBUF
        + [pltpu.SemaphoreType.DMA] * (2 * _NBUF)
    )
    out5 = pl.kernel(
        functools.partial(_k2_body, hist=hist),
        out_type=jax.ShapeDtypeStruct((hist, 8, n_bt, 8, _BT), jnp.float32),
        mesh=mesh,
        scratch_types=scratch,
        compiler_params=pltpu.CompilerParams(needs_layout_passes=False,
                                             use_tc_tiling_on_sc=False),
    )(xT, tbl)
    return out5.transpose(2, 4, 0, 1, 3).reshape(batch, hist, _D)


# speedup vs baseline: 1.0045x; 1.0045x over previous
import functools

import jax
import jax.numpy as jnp
from jax import lax
from jax.experimental import pallas as pl
from jax.experimental.pallas import tpu as pltpu
from jax.experimental.pallas import tpu_sc as plsc

_D = 64
_BT = 128
_NBUF = 4
_NC = 2
_NS = 16
_NW = _NC * _NS
_L = 16
_VPAD = 1000448
_ZROW = 1000000


def _k1_body(tin_ref, tout_ref):
    blk = tin_ref[...]
    t = blk.T
    r0 = pl.program_id(0) * 256
    rows = jax.lax.broadcasted_iota(jnp.int32, (256, _D), 0) + r0
    tm = jnp.where(rows < _ZROW, t, 0.0)
    tout_ref[...] = jnp.concatenate(
        [tm, jnp.zeros((256, _D), jnp.float32)], axis=1)


def _table_rowmajor(tT):
    return pl.pallas_call(
        _k1_body,
        grid=(3907,),
        in_specs=[pl.BlockSpec((64, 256), lambda i: (0, i))],
        out_specs=pl.BlockSpec((256, 128), lambda i: (i, 0)),
        out_shape=jax.ShapeDtypeStruct((_VPAD, 128), jnp.float32),
    )(tT)


def _k2_body(xT_hbm, tbl_hbm, out_hbm, idx_all, *rest, hist):
    pbufs = rest[:_NBUF]
    tbufs = rest[_NBUF:2 * _NBUF]
    gsems = rest[2 * _NBUF:3 * _NBUF]
    ssems = rest[3 * _NBUF:4 * _NBUF]

    wid = lax.axis_index("s") * _NC + lax.axis_index("c")

    pltpu.sync_copy(xT_hbm.at[:, pl.ds(wid * _BT, _BT)], idx_all)

    def remap(h, carry):
        for k in range(_BT // _L):
            iv = idx_all[h, pl.ds(k * _L, _L)]
            iv = jnp.where(iv == 0, _ZROW, iv) * 2
            idx_all[h, pl.ds(k * _L, _L)] = iv
        return carry

    lax.fori_loop(0, hist, remap, 0)

    def fire_gather(b, h):
        pltpu.async_copy(tbl_hbm.at[idx_all.at[h]], pbufs[b], gsems[b])

    def wait_gather(b, h):
        pltpu.make_async_copy(tbl_hbm.at[idx_all.at[h]], pbufs[b],
                              gsems[b]).wait()

    def fire_store(b, h):
        pltpu.async_copy(tbufs[b], out_hbm.at[h, :, wid], ssems[b])

    def wait_store(b, h):
        pltpu.make_async_copy(tbufs[b], out_hbm.at[h, :, wid],
                              ssems[b]).wait()

    base = lax.iota(jnp.int32, _L)
    rowvecs = [base + k * _L for k in range(_BT // _L)]

    def transpose(b):
        pbuf, tbuf = pbufs[b], tbufs[b]

        def dloop(d0, carry):
            colv = (base + d0) & 63
            dtv = colv >> 3
            dsv = colv & 7
            for k in range(_BT // _L):
                v = plsc.load_gather(pbuf, [rowvecs[k], colv])
                plsc.store_scatter(tbuf, [dtv, dsv, rowvecs[k]], v)
            return carry

        lax.fori_loop(0, _D, dloop, 0)

    for b in range(_NBUF):
        fire_gather(b, b)

    def outer(i, carry):
        for b in range(_NBUF):
            h = i * _NBUF + b
            wait_gather(b, h)
            transpose(b)
            fire_store(b, h)

            @pl.when(h + _NBUF < hist)
            def _():
                wait_store(b, h)
                fire_gather(b, h + _NBUF)
        return carry

    lax.fori_loop(0, hist // _NBUF, outer, 0)

    for b in range(_NBUF):
        wait_store(b, hist - _NBUF + b)


def kernel(X, table):
    batch, hist = X.shape
    n_bt = batch // _BT

    tT = table.T
    tbl = _table_rowmajor(tT).reshape(2 * _VPAD, _D)
    xT = X.T

    mesh = plsc.VectorSubcoreMesh(core_axis_name="c", subcore_axis_name="s",
                                  num_cores=_NC, num_subcores=_NS)
    scratch = (
        [pltpu.VMEM((hist, _BT), jnp.int32)]
        + [pltpu.VMEM((_BT, _D), jnp.float32)] * _NBUF
        + [pltpu.VMEM((8, 8, _BT), jnp.float32)] * _NBUF
        + [pltpu.SemaphoreType.DMA] * (2 * _NBUF)
    )
    out5 = pl.kernel(
        functools.partial(_k2_body, hist=hist),
        out_type=jax.ShapeDtypeStruct((hist, 8, n_bt, 8, _BT), jnp.float32),
        mesh=mesh,
        scratch_types=scratch,
        compiler_params=pltpu.CompilerParams(needs_layout_passes=False,
                                             use_tc_tiling_on_sc=False),
    )(xT, tbl)
    return out5.transpose(2, 4, 0, 1, 3).reshape(batch, hist, _D)


# speedup vs baseline: 3.1905x; 3.1763x over previous
import functools

import jax
import jax.numpy as jnp
from jax import lax
from jax.experimental import pallas as pl
from jax.experimental.pallas import tpu as pltpu
from jax.experimental.pallas import tpu_sc as plsc

_D = 64
_BT = 128
_NBUF = 4
_NC = 2
_NS = 16
_NW = _NC * _NS
_L = 16
_WK1 = 4096
_GK1 = 245
_VROWS = _GK1 * _WK1
_ZROW = 1000000


def _k1_body(tin_ref, tout_ref):
    blk = tin_ref[...]
    t = blk.T
    r0 = pl.program_id(0) * _WK1
    rows = jax.lax.broadcasted_iota(jnp.int32, (_WK1, _D), 0) + r0
    tm = jnp.where(rows < _ZROW, t, 0.0)
    tout_ref[...] = jnp.concatenate(
        [tm, jnp.zeros((_WK1, _D), jnp.float32)], axis=1)


def _table_rowmajor(tT):
    return pl.pallas_call(
        _k1_body,
        grid=(_GK1,),
        in_specs=[pl.BlockSpec((_D, _WK1), lambda i: (0, i))],
        out_specs=pl.BlockSpec((_WK1, 128), lambda i: (i, 0)),
        out_shape=jax.ShapeDtypeStruct((_VROWS, 128), jnp.float32),
    )(tT)


def _k2_body(xT_hbm, tbl_hbm, out_hbm, idx_all, *rest, hist):
    pbufs = rest[:_NBUF]
    tbufs = rest[_NBUF:2 * _NBUF]
    gsems = rest[2 * _NBUF:3 * _NBUF]
    ssems = rest[3 * _NBUF:4 * _NBUF]

    wid = lax.axis_index("s") * _NC + lax.axis_index("c")

    pltpu.sync_copy(xT_hbm.at[:, pl.ds(wid * _BT, _BT)], idx_all)

    def remap(h, carry):
        for k in range(_BT // _L):
            iv = idx_all[h, pl.ds(k * _L, _L)]
            iv = jnp.where(iv == 0, _ZROW, iv) * 2
            idx_all[h, pl.ds(k * _L, _L)] = iv
        return carry

    lax.fori_loop(0, hist, remap, 0)

    def fire_gather(b, h):
        pltpu.async_copy(tbl_hbm.at[idx_all.at[h]], pbufs[b], gsems[b])

    def wait_gather(b, h):
        pltpu.make_async_copy(tbl_hbm.at[idx_all.at[h]], pbufs[b],
                              gsems[b]).wait()

    def fire_store(b, h):
        pltpu.async_copy(tbufs[b], out_hbm.at[h, :, wid], ssems[b])

    def wait_store(b, h):
        pltpu.make_async_copy(tbufs[b], out_hbm.at[h, :, wid],
                              ssems[b]).wait()

    base = lax.iota(jnp.int32, _L)
    rowvecs = [base + k * _L for k in range(_BT // _L)]

    def transpose(b):
        pbuf, tbuf = pbufs[b], tbufs[b]

        def dloop(d0, carry):
            colv = (base + d0) & 63
            dtv = colv >> 3
            dsv = colv & 7
            for k in range(_BT // _L):
                v = plsc.load_gather(pbuf, [rowvecs[k], colv])
                plsc.store_scatter(tbuf, [dtv, dsv, rowvecs[k]], v)
            return carry

        lax.fori_loop(0, _D, dloop, 0)

    for b in range(_NBUF):
        fire_gather(b, b)

    def outer(i, carry):
        for b in range(_NBUF):
            h = i * _NBUF + b
            wait_gather(b, h)
            transpose(b)
            fire_store(b, h)

            @pl.when(h + _NBUF < hist)
            def _():
                wait_store(b, h)
                fire_gather(b, h + _NBUF)
        return carry

    lax.fori_loop(0, hist // _NBUF, outer, 0)

    for b in range(_NBUF):
        wait_store(b, hist - _NBUF + b)


def kernel(X, table):
    batch, hist = X.shape
    n_bt = batch // _BT

    tT = table.T
    tbl = _table_rowmajor(tT).reshape(2 * _VROWS, _D)
    xT = X.T

    mesh = plsc.VectorSubcoreMesh(core_axis_name="c", subcore_axis_name="s",
                                  num_cores=_NC, num_subcores=_NS)
    scratch = (
        [pltpu.VMEM((hist, _BT), jnp.int32)]
        + [pltpu.VMEM((_BT, _D), jnp.float32)] * _NBUF
        + [pltpu.VMEM((8, 8, _BT), jnp.float32)] * _NBUF
        + [pltpu.SemaphoreType.DMA] * (2 * _NBUF)
    )
    out5 = pl.kernel(
        functools.partial(_k2_body, hist=hist),
        out_type=jax.ShapeDtypeStruct((hist, 8, n_bt, 8, _BT), jnp.float32),
        mesh=mesh,
        scratch_types=scratch,
        compiler_params=pltpu.CompilerParams(needs_layout_passes=False,
                                             use_tc_tiling_on_sc=False),
    )(xT, tbl)
    return out5.transpose(2, 4, 0, 1, 3).reshape(batch, hist, _D)


# speedup vs baseline: 3.2294x; 1.0122x over previous
import functools

import jax
import jax.numpy as jnp
from jax import lax
from jax.experimental import pallas as pl
from jax.experimental.pallas import tpu as pltpu
from jax.experimental.pallas import tpu_sc as plsc

_D = 64
_BT = 128
_NBUF = 4
_NC = 2
_NS = 16
_NW = _NC * _NS
_L = 16
_WK1 = 4096
_GK1 = 245
_VROWS = _GK1 * _WK1
_ZROW = 1000000


def _k1_body(tin_ref, tout_ref):
    blk = tin_ref[...]
    r0 = pl.program_id(0) * _WK1
    cols = jax.lax.broadcasted_iota(jnp.int32, (_D, _WK1), 1) + r0
    bm = jnp.where(cols < _ZROW, blk, 0.0)
    lo = jax.lax.slice(bm, (0, 0), (_D, _WK1 // 2))
    hi = jax.lax.slice(bm, (0, _WK1 // 2), (_D, _WK1))
    tout_ref[...] = jnp.concatenate([lo.T, hi.T], axis=1)


def _table_rowmajor(tT):
    return pl.pallas_call(
        _k1_body,
        grid=(_GK1,),
        in_specs=[pl.BlockSpec((_D, _WK1), lambda i: (0, i))],
        out_specs=pl.BlockSpec((_WK1 // 2, 128), lambda i: (i, 0)),
        out_shape=jax.ShapeDtypeStruct((_VROWS // 2, 128), jnp.float32),
    )(tT)


def _k2_body(xT_hbm, tbl_hbm, out_hbm, idx_all, *rest, hist):
    pbufs = rest[:_NBUF]
    tbufs = rest[_NBUF:2 * _NBUF]
    gsems = rest[2 * _NBUF:3 * _NBUF]
    ssems = rest[3 * _NBUF:4 * _NBUF]

    wid = lax.axis_index("s") * _NC + lax.axis_index("c")

    pltpu.sync_copy(xT_hbm.at[:, pl.ds(wid * _BT, _BT)], idx_all)

    def remap(h, carry):
        for k in range(_BT // _L):
            iv = idx_all[h, pl.ds(k * _L, _L)]
            iv = jnp.where(iv == 0, _ZROW, iv)
            lo = iv & (_WK1 - 1)
            pos = jnp.where(lo < _WK1 // 2,
                            lo * 2, (lo - _WK1 // 2) * 2 + 1)
            idx_all[h, pl.ds(k * _L, _L)] = (iv - lo) + pos
        return carry

    lax.fori_loop(0, hist, remap, 0)

    def fire_gather(b, h):
        pltpu.async_copy(tbl_hbm.at[idx_all.at[h]], pbufs[b], gsems[b])

    def wait_gather(b, h):
        pltpu.make_async_copy(tbl_hbm.at[idx_all.at[h]], pbufs[b],
                              gsems[b]).wait()

    def fire_store(b, h):
        pltpu.async_copy(tbufs[b], out_hbm.at[h, :, wid], ssems[b])

    def wait_store(b, h):
        pltpu.make_async_copy(tbufs[b], out_hbm.at[h, :, wid],
                              ssems[b]).wait()

    base = lax.iota(jnp.int32, _L)
    rowvecs = [base + k * _L for k in range(_BT // _L)]

    def transpose(b):
        pbuf, tbuf = pbufs[b], tbufs[b]

        def dloop(d0, carry):
            colv = (base + d0) & 63
            dtv = colv >> 3
            dsv = colv & 7
            for k in range(_BT // _L):
                v = plsc.load_gather(pbuf, [rowvecs[k], colv])
                plsc.store_scatter(tbuf, [dtv, dsv, rowvecs[k]], v)
            return carry

        lax.fori_loop(0, _D, dloop, 0)

    for b in range(_NBUF):
        fire_gather(b, b)

    def outer(i, carry):
        for b in range(_NBUF):
            h = i * _NBUF + b
            wait_gather(b, h)
            transpose(b)
            fire_store(b, h)

            @pl.when(h + _NBUF < hist)
            def _():
                wait_store(b, h)
                fire_gather(b, h + _NBUF)
        return carry

    lax.fori_loop(0, hist // _NBUF, outer, 0)

    for b in range(_NBUF):
        wait_store(b, hist - _NBUF + b)


def kernel(X, table):
    batch, hist = X.shape
    n_bt = batch // _BT

    tT = table.T
    tbl = _table_rowmajor(tT).reshape(_VROWS, _D)
    xT = X.T

    mesh = plsc.VectorSubcoreMesh(core_axis_name="c", subcore_axis_name="s",
                                  num_cores=_NC, num_subcores=_NS)
    scratch = (
        [pltpu.VMEM((hist, _BT), jnp.int32)]
        + [pltpu.VMEM((_BT, _D), jnp.float32)] * _NBUF
        + [pltpu.VMEM((8, 8, _BT), jnp.float32)] * _NBUF
        + [pltpu.SemaphoreType.DMA] * (2 * _NBUF)
    )
    out5 = pl.kernel(
        functools.partial(_k2_body, hist=hist),
        out_type=jax.ShapeDtypeStruct((hist, 8, n_bt, 8, _BT), jnp.float32),
        mesh=mesh,
        scratch_types=scratch,
        compiler_params=pltpu.CompilerParams(needs_layout_passes=False,
                                             use_tc_tiling_on_sc=False),
    )(xT, tbl)
    return out5.transpose(2, 4, 0, 1, 3).reshape(batch, hist, _D)
